# baseline (speedup 1.0000x reference)
import functools

import jax
import jax.numpy as jnp
from jax import lax
from jax.experimental import pallas as pl
from jax.experimental.pallas import tpu as pltpu
from jax.experimental.pallas import tpu_sc as plsc

NUM_CLASSES = 256
FEATURE_DIM = 512
N_ROWS = 32768
EPS = 1e-08

NUM_CORES = 2
NUM_SUBCORES = 16
NUM_WORKERS = NUM_CORES * NUM_SUBCORES

SC_ROWS = 8192
TC_ROWS = N_ROWS - SC_ROWS
TC_BLK = 1024

NUM_CB = 4
NUM_RG = NUM_WORKERS // NUM_CB
CB_W = FEATURE_DIM // NUM_CB
RG_ROWS = SC_ROWS // NUM_RG
CHUNK = 128
NUM_CHUNKS = RG_ROWS // CHUNK
NBUF = 2
NUM_STEPS = NUM_CHUNKS // NBUF
VPR = CB_W // 16

_GDN = lax.GatherDimensionNumbers(
    offset_dims=(), collapsed_slice_dims=(0,), start_index_map=(0,))


def _lane_bcast(vec16, i):
    return lax.gather(vec16, jnp.full((16, 1), i, jnp.int32), _GDN,
                      slice_sizes=(1,),
                      mode=lax.GatherScatterMode.PROMISE_IN_BOUNDS)


def _sc_segment_sums(features, labels32):
    mesh = plsc.VectorSubcoreMesh(core_axis_name="c", subcore_axis_name="s")

    @functools.partial(
        pl.kernel,
        out_type=(
            jax.ShapeDtypeStruct((NUM_RG, NUM_CLASSES, FEATURE_DIM), jnp.float32),
            jax.ShapeDtypeStruct((NUM_WORKERS, NUM_CLASSES, 16), jnp.float32),
        ),
        mesh=mesh,
        compiler_params=pltpu.CompilerParams(needs_layout_passes=False),
        scratch_types=[
            pltpu.VMEM((NBUF, CHUNK), jnp.int32),
            pltpu.VMEM((NBUF, CHUNK, CB_W), jnp.float32),
            pltpu.VMEM((NUM_CLASSES, CB_W), jnp.float32),
            pltpu.VMEM((NUM_CLASSES, 16), jnp.float32),
            [pltpu.SemaphoreType.DMA] * NBUF,
            [pltpu.SemaphoreType.DMA] * NBUF,
        ],
    )
    def seg(feat_hbm, lab_hbm, sums_out, cnts_out, lab_v, buf, acc, cnt_acc,
            sem_lab, sem_feat):
        c = lax.axis_index("c")
        s = lax.axis_index("s")
        wid = s * NUM_CORES + c
        rg = wid // NUM_CB
        cb = wid % NUM_CB
        r_base = rg * RG_ROWS
        c0 = cb * CB_W

        zeros16 = jnp.zeros((16,), jnp.float32)
        ones16 = jnp.ones((16,), jnp.float32)
        iota16 = lax.iota(jnp.int32, 16)

        def _zero(i, carry):
            for j in range(VPR):
                acc[i, pl.ds(j * 16, 16)] = zeros16
            cnt_acc[i, :] = zeros16
            return carry
        lax.fori_loop(0, NUM_CLASSES, _zero, 0)

        def _copies(k, b):
            r0 = r_base + k * CHUNK
            return (
                pltpu.make_async_copy(
                    lab_hbm.at[pl.ds(r0, CHUNK)], lab_v.at[b], sem_lab[b]),
                pltpu.make_async_copy(
                    feat_hbm.at[pl.ds(r0, CHUNK), pl.ds(c0, CB_W)], buf.at[b],
                    sem_feat[b]),
            )

        for b in range(NBUF):
            for cp in _copies(b, b):
                cp.start()

        def _process_chunk(k, b):
            def _group(g, carry):
                labels16 = lab_v[b, pl.ds(g * 16, 16)]

                def _row_loads(i):
                    return [buf[b, g * 16 + i, pl.ds(j * 16, 16)]
                            for j in range(VPR)]

                vals = _row_loads(0)
                for i in range(16):
                    bcast = _lane_bcast(labels16, i)
                    nxt = _row_loads(i + 1) if i < 15 else None
                    for j in range(VPR):
                        plsc.addupdate_scatter(acc, [bcast, iota16 + j * 16], vals[j])
                    vals = nxt
                return carry
            lax.fori_loop(0, CHUNK // 16, _group, 0)

            @pl.when(k // (NUM_CHUNKS // NUM_CB) == cb)
            def _count():
                def _cgroup(g, carry):
                    labels16 = lab_v[b, pl.ds(g * 16, 16)]
                    for i in range(16):
                        bcast = _lane_bcast(labels16, i)
                        plsc.addupdate_scatter(
                            cnt_acc, [bcast, iota16], ones16)
                    return carry
                lax.fori_loop(0, CHUNK // 16, _cgroup, 0)

        def _step(t, carry):
            for b in range(NBUF):
                k = NBUF * t + b
                for cp in _copies(k, b):
                    cp.wait()
                _process_chunk(k, b)
                @pl.when(k + NBUF < NUM_CHUNKS)
                def _refill(k=k, b=b):
                    for cp in _copies(k + NBUF, b):
                        cp.start()
            return carry
        lax.fori_loop(0, NUM_STEPS, _step, 0)

        pltpu.sync_copy(acc, sums_out.at[rg, :, pl.ds(c0, CB_W)])
        pltpu.sync_copy(cnt_acc, cnts_out.at[wid])

    return seg(features, labels32)


def _tc_seg_body(lab_ref, feat_ref, sums_ref, cnts_ref):
    step = pl.program_id(0)

    @pl.when(step == 0)
    def _init():
        sums_ref[...] = jnp.zeros_like(sums_ref)
        cnts_ref[...] = jnp.zeros_like(cnts_ref)

    labels = lab_ref[0, 0, :]
    onehot = (labels[None, :]
              == lax.broadcasted_iota(jnp.int32, (NUM_CLASSES, TC_BLK), 0)
              ).astype(jnp.float32)
    feat = feat_ref[...]
    sums_ref[...] += lax.dot_general(onehot, feat, (((1,), (0,)), ((), ())),
                                     preferred_element_type=jnp.float32)
    cnts_ref[...] += jnp.sum(onehot, axis=1, keepdims=True)


def _tc_segment_sums(features, labels32):
    sc_blocks = SC_ROWS // TC_BLK
    labels3 = labels32.reshape(N_ROWS // TC_BLK, 1, TC_BLK)
    return pl.pallas_call(
        _tc_seg_body,
        grid=(TC_ROWS // TC_BLK,),
        in_specs=[
            pl.BlockSpec((1, 1, TC_BLK), lambda i: (sc_blocks + i, 0, 0)),
            pl.BlockSpec((TC_BLK, FEATURE_DIM), lambda i: (sc_blocks + i, 0)),
        ],
        out_specs=[
            pl.BlockSpec((NUM_CLASSES, FEATURE_DIM), lambda i: (0, 0)),
            pl.BlockSpec((NUM_CLASSES, 1), lambda i: (0, 0)),
        ],
        out_shape=[
            jax.ShapeDtypeStruct((NUM_CLASSES, FEATURE_DIM), jnp.float32),
            jax.ShapeDtypeStruct((NUM_CLASSES, 1), jnp.float32),
        ],
    )(labels3, features)


def _tc_loss_body(sums_ref, cnts_ref, tc_sums_ref, tc_cnts_ref, out_ref):
    sums = jnp.sum(sums_ref[...], axis=0) + tc_sums_ref[...]
    cnt_all = jnp.sum(cnts_ref[...], axis=0)
    cnt = cnt_all[:, 0:1] + tc_cnts_ref[...]
    present = cnt > 0.0
    cent = jnp.where(present, sums / jnp.maximum(cnt, 1.0), 0.0)
    gram = lax.dot_general(cent, cent, (((1,), (1,)), ((), ())),
                           preferred_element_type=jnp.float32)
    norms = jnp.sum(cent * cent, axis=1, keepdims=True)
    d2 = norms + norms.reshape(1, NUM_CLASSES) - 2.0 * gram
    row = lax.broadcasted_iota(jnp.int32, (NUM_CLASSES, NUM_CLASSES), 0)
    col = lax.broadcasted_iota(jnp.int32, (NUM_CLASSES, NUM_CLASSES), 1)
    valid = (row < col) & present & present.reshape(1, NUM_CLASSES)
    safe = jnp.where(valid, jnp.maximum(d2, 0.0), 1.0)
    terms = jnp.where(valid, jnp.exp(-(jnp.sqrt(safe) / 16.0 + EPS)), 0.0)
    out_ref[...] = jnp.sum(terms)[None, None]


def kernel(features, labels):
    labels32 = labels.astype(jnp.int32)
    sums_p, cnts = _sc_segment_sums(features, labels32)
    tc_sums, tc_cnts = _tc_segment_sums(features, labels32)
    loss = pl.pallas_call(
        _tc_loss_body,
        out_shape=jax.ShapeDtypeStruct((1, 1), jnp.float32),
    )(sums_p, cnts, tc_sums, tc_cnts)
    return loss[0, 0]


# speedup vs baseline: 1.0353x; 1.0353x over previous
import functools

import jax
import jax.numpy as jnp
from jax import lax
from jax.experimental import pallas as pl
from jax.experimental.pallas import tpu as pltpu
from jax.experimental.pallas import tpu_sc as plsc

NUM_CLASSES = 256
FEATURE_DIM = 512
N_ROWS = 32768
EPS = 1e-08

NUM_CORES = 2
NUM_SUBCORES = 16
NUM_WORKERS = NUM_CORES * NUM_SUBCORES

SC_ROWS = 10240
TC_ROWS = N_ROWS - SC_ROWS
TC_BLK = 1024

NUM_CB = 4
NUM_RG = NUM_WORKERS // NUM_CB
CB_W = FEATURE_DIM // NUM_CB
RG_ROWS = SC_ROWS // NUM_RG
CHUNK = 128
NUM_CHUNKS = RG_ROWS // CHUNK
NBUF = 2
NUM_STEPS = NUM_CHUNKS // NBUF
VPR = CB_W // 16

_GDN = lax.GatherDimensionNumbers(
    offset_dims=(), collapsed_slice_dims=(0,), start_index_map=(0,))


def _lane_bcast(vec16, i):
    return lax.gather(vec16, jnp.full((16, 1), i, jnp.int32), _GDN,
                      slice_sizes=(1,),
                      mode=lax.GatherScatterMode.PROMISE_IN_BOUNDS)


def _sc_segment_sums(features, labels32):
    mesh = plsc.VectorSubcoreMesh(core_axis_name="c", subcore_axis_name="s")

    @functools.partial(
        pl.kernel,
        out_type=(
            jax.ShapeDtypeStruct((NUM_RG, NUM_CLASSES, FEATURE_DIM), jnp.float32),
            jax.ShapeDtypeStruct((NUM_WORKERS, NUM_CLASSES, 16), jnp.float32),
        ),
        mesh=mesh,
        compiler_params=pltpu.CompilerParams(needs_layout_passes=False),
        scratch_types=[
            pltpu.VMEM((NBUF, CHUNK), jnp.int32),
            pltpu.VMEM((NBUF, CHUNK, CB_W), jnp.float32),
            pltpu.VMEM((NUM_CLASSES, CB_W), jnp.float32),
            pltpu.VMEM((NUM_CLASSES, 16), jnp.float32),
            [pltpu.SemaphoreType.DMA] * NBUF,
            [pltpu.SemaphoreType.DMA] * NBUF,
        ],
    )
    def seg(feat_hbm, lab_hbm, sums_out, cnts_out, lab_v, buf, acc, cnt_acc,
            sem_lab, sem_feat):
        c = lax.axis_index("c")
        s = lax.axis_index("s")
        wid = s * NUM_CORES + c
        rg = wid // NUM_CB
        cb = wid % NUM_CB
        r_base = rg * RG_ROWS
        c0 = cb * CB_W

        zeros16 = jnp.zeros((16,), jnp.float32)
        ones16 = jnp.ones((16,), jnp.float32)
        iota16 = lax.iota(jnp.int32, 16)

        def _zero(i, carry):
            for j in range(VPR):
                acc[i, pl.ds(j * 16, 16)] = zeros16
            cnt_acc[i, :] = zeros16
            return carry
        lax.fori_loop(0, NUM_CLASSES, _zero, 0)

        def _copies(k, b):
            r0 = r_base + k * CHUNK
            return (
                pltpu.make_async_copy(
                    lab_hbm.at[pl.ds(r0, CHUNK)], lab_v.at[b], sem_lab[b]),
                pltpu.make_async_copy(
                    feat_hbm.at[pl.ds(r0, CHUNK), pl.ds(c0, CB_W)], buf.at[b],
                    sem_feat[b]),
            )

        for b in range(NBUF):
            for cp in _copies(b, b):
                cp.start()

        def _process_chunk(k, b):
            def _group(g, carry):
                labels16 = lab_v[b, pl.ds(g * 16, 16)]

                def _row_loads(i):
                    return [buf[b, g * 16 + i, pl.ds(j * 16, 16)]
                            for j in range(VPR)]

                vals = _row_loads(0)
                for i in range(16):
                    bcast = _lane_bcast(labels16, i)
                    nxt = _row_loads(i + 1) if i < 15 else None
                    for j in range(VPR):
                        plsc.addupdate_scatter(acc, [bcast, iota16 + j * 16], vals[j])
                    vals = nxt
                return carry
            lax.fori_loop(0, CHUNK // 16, _group, 0)

            @pl.when(k % NUM_CB == cb)
            def _count():
                def _cgroup(g, carry):
                    labels16 = lab_v[b, pl.ds(g * 16, 16)]
                    for i in range(16):
                        bcast = _lane_bcast(labels16, i)
                        plsc.addupdate_scatter(
                            cnt_acc, [bcast, iota16], ones16)
                    return carry
                lax.fori_loop(0, CHUNK // 16, _cgroup, 0)

        def _step(t, carry):
            for b in range(NBUF):
                k = NBUF * t + b
                for cp in _copies(k, b):
                    cp.wait()
                _process_chunk(k, b)
                @pl.when(k + NBUF < NUM_CHUNKS)
                def _refill(k=k, b=b):
                    for cp in _copies(k + NBUF, b):
                        cp.start()
            return carry
        lax.fori_loop(0, NUM_STEPS, _step, 0)

        pltpu.sync_copy(acc, sums_out.at[rg, :, pl.ds(c0, CB_W)])
        pltpu.sync_copy(cnt_acc, cnts_out.at[wid])

    return seg(features, labels32)


def _tc_seg_body(lab_ref, feat_ref, sums_ref, cnts_ref):
    step = pl.program_id(0)

    @pl.when(step == 0)
    def _init():
        sums_ref[...] = jnp.zeros_like(sums_ref)
        cnts_ref[...] = jnp.zeros_like(cnts_ref)

    labels = lab_ref[0, 0, :]
    onehot = (labels[None, :]
              == lax.broadcasted_iota(jnp.int32, (NUM_CLASSES, TC_BLK), 0)
              ).astype(jnp.float32)
    feat = feat_ref[...]
    sums_ref[...] += lax.dot_general(onehot, feat, (((1,), (0,)), ((), ())),
                                     preferred_element_type=jnp.float32)
    cnts_ref[...] += jnp.sum(onehot, axis=1, keepdims=True)


def _tc_segment_sums(features, labels32):
    sc_blocks = SC_ROWS // TC_BLK
    labels3 = labels32.reshape(N_ROWS // TC_BLK, 1, TC_BLK)
    return pl.pallas_call(
        _tc_seg_body,
        grid=(TC_ROWS // TC_BLK,),
        in_specs=[
            pl.BlockSpec((1, 1, TC_BLK), lambda i: (sc_blocks + i, 0, 0)),
            pl.BlockSpec((TC_BLK, FEATURE_DIM), lambda i: (sc_blocks + i, 0)),
        ],
        out_specs=[
            pl.BlockSpec((NUM_CLASSES, FEATURE_DIM), lambda i: (0, 0)),
            pl.BlockSpec((NUM_CLASSES, 1), lambda i: (0, 0)),
        ],
        out_shape=[
            jax.ShapeDtypeStruct((NUM_CLASSES, FEATURE_DIM), jnp.float32),
            jax.ShapeDtypeStruct((NUM_CLASSES, 1), jnp.float32),
        ],
    )(labels3, features)


def _tc_loss_body(sums_ref, cnts_ref, tc_sums_ref, tc_cnts_ref, out_ref):
    sums = jnp.sum(sums_ref[...], axis=0) + tc_sums_ref[...]
    cnt_all = jnp.sum(cnts_ref[...], axis=0)
    cnt = cnt_all[:, 0:1] + tc_cnts_ref[...]
    present = cnt > 0.0
    cent = jnp.where(present, sums / jnp.maximum(cnt, 1.0), 0.0)
    gram = lax.dot_general(cent, cent, (((1,), (1,)), ((), ())),
                           preferred_element_type=jnp.float32)
    norms = jnp.sum(cent * cent, axis=1, keepdims=True)
    d2 = norms + norms.reshape(1, NUM_CLASSES) - 2.0 * gram
    row = lax.broadcasted_iota(jnp.int32, (NUM_CLASSES, NUM_CLASSES), 0)
    col = lax.broadcasted_iota(jnp.int32, (NUM_CLASSES, NUM_CLASSES), 1)
    valid = (row < col) & present & present.reshape(1, NUM_CLASSES)
    safe = jnp.where(valid, jnp.maximum(d2, 0.0), 1.0)
    terms = jnp.where(valid, jnp.exp(-(jnp.sqrt(safe) / 16.0 + EPS)), 0.0)
    out_ref[...] = jnp.sum(terms)[None, None]


def kernel(features, labels):
    labels32 = labels.astype(jnp.int32)
    sums_p, cnts = _sc_segment_sums(features, labels32)
    tc_sums, tc_cnts = _tc_segment_sums(features, labels32)
    loss = pl.pallas_call(
        _tc_loss_body,
        out_shape=jax.ShapeDtypeStruct((1, 1), jnp.float32),
    )(sums_p, cnts, tc_sums, tc_cnts)
    return loss[0, 0]


# speedup vs baseline: 1.0390x; 1.0036x over previous
import functools

import jax
import jax.numpy as jnp
from jax import lax
from jax.experimental import pallas as pl
from jax.experimental.pallas import tpu as pltpu
from jax.experimental.pallas import tpu_sc as plsc

NUM_CLASSES = 256
FEATURE_DIM = 512
N_ROWS = 32768
EPS = 1e-08

NUM_CORES = 2
NUM_SUBCORES = 16
NUM_WORKERS = NUM_CORES * NUM_SUBCORES

SC_ROWS = 10240
TC_ROWS = N_ROWS - SC_ROWS
TC_BLK = 1024

NUM_CB = 4
NUM_RG = NUM_WORKERS // NUM_CB
CB_W = FEATURE_DIM // NUM_CB
RG_ROWS = SC_ROWS // NUM_RG
CHUNK = 128
NUM_CHUNKS = RG_ROWS // CHUNK
NBUF = 2
NUM_STEPS = NUM_CHUNKS // NBUF
VPR = CB_W // 16

_GDN = lax.GatherDimensionNumbers(
    offset_dims=(), collapsed_slice_dims=(0,), start_index_map=(0,))


def _lane_bcast(vec16, i):
    return lax.gather(vec16, jnp.full((16, 1), i, jnp.int32), _GDN,
                      slice_sizes=(1,),
                      mode=lax.GatherScatterMode.PROMISE_IN_BOUNDS)


def _sc_segment_sums(features, labels32):
    mesh = plsc.VectorSubcoreMesh(core_axis_name="c", subcore_axis_name="s")

    @functools.partial(
        pl.kernel,
        out_type=(
            jax.ShapeDtypeStruct((NUM_RG, NUM_CLASSES, FEATURE_DIM), jnp.float32),
            jax.ShapeDtypeStruct((NUM_WORKERS, NUM_CLASSES, 16), jnp.float32),
        ),
        mesh=mesh,
        compiler_params=pltpu.CompilerParams(needs_layout_passes=False),
        scratch_types=[
            pltpu.VMEM((NBUF, CHUNK), jnp.int32),
            pltpu.VMEM((NBUF, CHUNK, CB_W), jnp.float32),
            pltpu.VMEM((NUM_CLASSES, CB_W), jnp.float32),
            pltpu.VMEM((NUM_CLASSES, 16), jnp.float32),
            [pltpu.SemaphoreType.DMA] * NBUF,
            [pltpu.SemaphoreType.DMA] * NBUF,
        ],
    )
    def seg(feat_hbm, lab_hbm, sums_out, cnts_out, lab_v, buf, acc, cnt_acc,
            sem_lab, sem_feat):
        c = lax.axis_index("c")
        s = lax.axis_index("s")
        wid = s * NUM_CORES + c
        rg = wid // NUM_CB
        cb = wid % NUM_CB
        r_base = rg * RG_ROWS
        c0 = cb * CB_W

        zeros16 = jnp.zeros((16,), jnp.float32)
        ones16 = jnp.ones((16,), jnp.float32)
        iota16 = lax.iota(jnp.int32, 16)

        def _zero(i, carry):
            for j in range(VPR):
                acc[i, pl.ds(j * 16, 16)] = zeros16
            cnt_acc[i, :] = zeros16
            return carry
        lax.fori_loop(0, NUM_CLASSES, _zero, 0)

        def _copies(k, b):
            r0 = r_base + k * CHUNK
            return (
                pltpu.make_async_copy(
                    lab_hbm.at[pl.ds(r0, CHUNK)], lab_v.at[b], sem_lab[b]),
                pltpu.make_async_copy(
                    feat_hbm.at[pl.ds(r0, CHUNK), pl.ds(c0, CB_W)], buf.at[b],
                    sem_feat[b]),
            )

        for b in range(NBUF):
            for cp in _copies(b, b):
                cp.start()

        def _process_chunk(k, b):
            def _group(g, carry):
                labels16 = lab_v[b, pl.ds(g * 16, 16)]

                def _row_loads(i):
                    return [buf[b, g * 16 + i, pl.ds(j * 16, 16)]
                            for j in range(VPR)]

                vals = _row_loads(0)
                for i in range(16):
                    bcast = _lane_bcast(labels16, i)
                    nxt = _row_loads(i + 1) if i < 15 else None
                    for j in range(VPR):
                        plsc.addupdate_scatter(acc, [bcast, iota16 + j * 16], vals[j])
                    vals = nxt
                return carry
            lax.fori_loop(0, CHUNK // 16, _group, 0)

            @pl.when(k % NUM_CB == cb)
            def _count():
                def _cgroup(g, carry):
                    labels16 = lab_v[b, pl.ds(g * 16, 16)]
                    for i in range(16):
                        bcast = _lane_bcast(labels16, i)
                        plsc.addupdate_scatter(
                            cnt_acc, [bcast, iota16], ones16)
                    return carry
                lax.fori_loop(0, CHUNK // 16, _cgroup, 0)

        def _step(t, carry):
            for b in range(NBUF):
                k = NBUF * t + b
                for cp in _copies(k, b):
                    cp.wait()
                _process_chunk(k, b)
                @pl.when(k + NBUF < NUM_CHUNKS)
                def _refill(k=k, b=b):
                    for cp in _copies(k + NBUF, b):
                        cp.start()
            return carry
        lax.fori_loop(0, NUM_STEPS, _step, 0)

        pltpu.sync_copy(acc, sums_out.at[rg, :, pl.ds(c0, CB_W)])
        pltpu.sync_copy(cnt_acc, cnts_out.at[wid])

    return seg(features, labels32)


def _tc_seg_body(lab_ref, feat_ref, sums_ref, cnts_ref):
    step = pl.program_id(0)

    @pl.when(step == 0)
    def _init():
        sums_ref[...] = jnp.zeros_like(sums_ref)
        cnts_ref[...] = jnp.zeros_like(cnts_ref)

    labels = lab_ref[0, 0, :]
    onehot = (labels[None, :]
              == lax.broadcasted_iota(jnp.int32, (NUM_CLASSES, TC_BLK), 0)
              ).astype(jnp.float32)
    feat = feat_ref[...].astype(jnp.bfloat16)
    sums_ref[...] += lax.dot_general(onehot.astype(jnp.bfloat16), feat,
                                     (((1,), (0,)), ((), ())),
                                     preferred_element_type=jnp.float32)
    cnts_ref[...] += jnp.sum(onehot, axis=1, keepdims=True)


def _tc_segment_sums(features, labels32):
    sc_blocks = SC_ROWS // TC_BLK
    labels3 = labels32.reshape(N_ROWS // TC_BLK, 1, TC_BLK)
    return pl.pallas_call(
        _tc_seg_body,
        grid=(TC_ROWS // TC_BLK,),
        in_specs=[
            pl.BlockSpec((1, 1, TC_BLK), lambda i: (sc_blocks + i, 0, 0)),
            pl.BlockSpec((TC_BLK, FEATURE_DIM), lambda i: (sc_blocks + i, 0)),
        ],
        out_specs=[
            pl.BlockSpec((NUM_CLASSES, FEATURE_DIM), lambda i: (0, 0)),
            pl.BlockSpec((NUM_CLASSES, 1), lambda i: (0, 0)),
        ],
        out_shape=[
            jax.ShapeDtypeStruct((NUM_CLASSES, FEATURE_DIM), jnp.float32),
            jax.ShapeDtypeStruct((NUM_CLASSES, 1), jnp.float32),
        ],
    )(labels3, features)


def _tc_loss_body(sums_ref, cnts_ref, tc_sums_ref, tc_cnts_ref, out_ref):
    sums = jnp.sum(sums_ref[...], axis=0) + tc_sums_ref[...]
    cnt_all = jnp.sum(cnts_ref[...], axis=0)
    cnt = cnt_all[:, 0:1] + tc_cnts_ref[...]
    present = cnt > 0.0
    cent = jnp.where(present, sums / jnp.maximum(cnt, 1.0), 0.0)
    gram = lax.dot_general(cent, cent, (((1,), (1,)), ((), ())),
                           preferred_element_type=jnp.float32)
    norms = jnp.sum(cent * cent, axis=1, keepdims=True)
    d2 = norms + norms.reshape(1, NUM_CLASSES) - 2.0 * gram
    row = lax.broadcasted_iota(jnp.int32, (NUM_CLASSES, NUM_CLASSES), 0)
    col = lax.broadcasted_iota(jnp.int32, (NUM_CLASSES, NUM_CLASSES), 1)
    valid = (row < col) & present & present.reshape(1, NUM_CLASSES)
    safe = jnp.where(valid, jnp.maximum(d2, 0.0), 1.0)
    terms = jnp.where(valid, jnp.exp(-(jnp.sqrt(safe) / 16.0 + EPS)), 0.0)
    out_ref[...] = jnp.sum(terms)[None, None]


def kernel(features, labels):
    labels32 = labels.astype(jnp.int32)
    sums_p, cnts = _sc_segment_sums(features, labels32)
    tc_sums, tc_cnts = _tc_segment_sums(features, labels32)
    loss = pl.pallas_call(
        _tc_loss_body,
        out_shape=jax.ShapeDtypeStruct((1, 1), jnp.float32),
    )(sums_p, cnts, tc_sums, tc_cnts)
    return loss[0, 0]


# speedup vs baseline: 1.0480x; 1.0087x over previous
import functools

import jax
import jax.numpy as jnp
from jax import lax
from jax.experimental import pallas as pl
from jax.experimental.pallas import tpu as pltpu
from jax.experimental.pallas import tpu_sc as plsc

NUM_CLASSES = 256
FEATURE_DIM = 512
N_ROWS = 32768
EPS = 1e-08

NUM_CORES = 2
NUM_SUBCORES = 16
NUM_WORKERS = NUM_CORES * NUM_SUBCORES

SC_ROWS = 12288
TC_ROWS = N_ROWS - SC_ROWS
TC_BLK = 1024

NUM_CB = 4
NUM_RG = NUM_WORKERS // NUM_CB
CB_W = FEATURE_DIM // NUM_CB
RG_ROWS = SC_ROWS // NUM_RG
CHUNK = 128
NUM_CHUNKS = RG_ROWS // CHUNK
NBUF = 3
NUM_STEPS = NUM_CHUNKS // NBUF
VPR = CB_W // 16

_GDN = lax.GatherDimensionNumbers(
    offset_dims=(), collapsed_slice_dims=(0,), start_index_map=(0,))


def _lane_bcast(vec16, i):
    return lax.gather(vec16, jnp.full((16, 1), i, jnp.int32), _GDN,
                      slice_sizes=(1,),
                      mode=lax.GatherScatterMode.PROMISE_IN_BOUNDS)


def _sc_segment_sums(features, labels32):
    mesh = plsc.VectorSubcoreMesh(core_axis_name="c", subcore_axis_name="s")

    @functools.partial(
        pl.kernel,
        out_type=(
            jax.ShapeDtypeStruct((NUM_RG, NUM_CLASSES, FEATURE_DIM), jnp.float32),
            jax.ShapeDtypeStruct((NUM_WORKERS, NUM_CLASSES, 16), jnp.float32),
        ),
        mesh=mesh,
        compiler_params=pltpu.CompilerParams(needs_layout_passes=False),
        scratch_types=[
            pltpu.VMEM((NBUF, CHUNK), jnp.int32),
            pltpu.VMEM((NBUF, CHUNK, CB_W), jnp.float32),
            pltpu.VMEM((NUM_CLASSES, CB_W), jnp.float32),
            pltpu.VMEM((NUM_CLASSES, 16), jnp.float32),
            [pltpu.SemaphoreType.DMA] * NBUF,
            [pltpu.SemaphoreType.DMA] * NBUF,
        ],
    )
    def seg(feat_hbm, lab_hbm, sums_out, cnts_out, lab_v, buf, acc, cnt_acc,
            sem_lab, sem_feat):
        c = lax.axis_index("c")
        s = lax.axis_index("s")
        wid = s * NUM_CORES + c
        rg = wid // NUM_CB
        cb = wid % NUM_CB
        r_base = rg * RG_ROWS
        c0 = cb * CB_W

        zeros16 = jnp.zeros((16,), jnp.float32)
        ones16 = jnp.ones((16,), jnp.float32)
        iota16 = lax.iota(jnp.int32, 16)

        def _zero(i, carry):
            for j in range(VPR):
                acc[i, pl.ds(j * 16, 16)] = zeros16
            cnt_acc[i, :] = zeros16
            return carry
        lax.fori_loop(0, NUM_CLASSES, _zero, 0)

        def _copies(k, b):
            r0 = r_base + k * CHUNK
            return (
                pltpu.make_async_copy(
                    lab_hbm.at[pl.ds(r0, CHUNK)], lab_v.at[b], sem_lab[b]),
                pltpu.make_async_copy(
                    feat_hbm.at[pl.ds(r0, CHUNK), pl.ds(c0, CB_W)], buf.at[b],
                    sem_feat[b]),
            )

        for b in range(NBUF):
            for cp in _copies(b, b):
                cp.start()

        def _process_chunk(k, b):
            def _group(g, carry):
                labels16 = lab_v[b, pl.ds(g * 16, 16)]

                def _row_loads(i):
                    return [buf[b, g * 16 + i, pl.ds(j * 16, 16)]
                            for j in range(VPR)]

                vals = _row_loads(0)
                for i in range(16):
                    bcast = _lane_bcast(labels16, i)
                    nxt = _row_loads(i + 1) if i < 15 else None
                    for j in range(VPR):
                        plsc.addupdate_scatter(acc, [bcast, iota16 + j * 16], vals[j])
                    vals = nxt
                return carry
            lax.fori_loop(0, CHUNK // 16, _group, 0)

            @pl.when(k % NUM_CB == cb)
            def _count():
                def _cgroup(g, carry):
                    labels16 = lab_v[b, pl.ds(g * 16, 16)]
                    for i in range(16):
                        bcast = _lane_bcast(labels16, i)
                        plsc.addupdate_scatter(
                            cnt_acc, [bcast, iota16], ones16)
                    return carry
                lax.fori_loop(0, CHUNK // 16, _cgroup, 0)

        def _step(t, carry):
            for b in range(NBUF):
                k = NBUF * t + b
                for cp in _copies(k, b):
                    cp.wait()
                _process_chunk(k, b)
                @pl.when(k + NBUF < NUM_CHUNKS)
                def _refill(k=k, b=b):
                    for cp in _copies(k + NBUF, b):
                        cp.start()
            return carry
        lax.fori_loop(0, NUM_STEPS, _step, 0)

        pltpu.sync_copy(acc, sums_out.at[rg, :, pl.ds(c0, CB_W)])
        pltpu.sync_copy(cnt_acc, cnts_out.at[wid])

    return seg(features, labels32)


def _tc_seg_body(lab_ref, feat_ref, sums_ref, cnts_ref):
    step = pl.program_id(0)

    @pl.when(step == 0)
    def _init():
        sums_ref[...] = jnp.zeros_like(sums_ref)
        cnts_ref[...] = jnp.zeros_like(cnts_ref)

    labels = lab_ref[0, 0, :]
    onehot = (labels[None, :]
              == lax.broadcasted_iota(jnp.int32, (NUM_CLASSES, TC_BLK), 0)
              ).astype(jnp.float32)
    feat = feat_ref[...].astype(jnp.bfloat16)
    sums_ref[...] += lax.dot_general(onehot.astype(jnp.bfloat16), feat,
                                     (((1,), (0,)), ((), ())),
                                     preferred_element_type=jnp.float32)
    cnts_ref[...] += jnp.sum(onehot, axis=1, keepdims=True)


def _tc_segment_sums(features, labels32):
    sc_blocks = SC_ROWS // TC_BLK
    labels3 = labels32.reshape(N_ROWS // TC_BLK, 1, TC_BLK)
    return pl.pallas_call(
        _tc_seg_body,
        grid=(TC_ROWS // TC_BLK,),
        in_specs=[
            pl.BlockSpec((1, 1, TC_BLK), lambda i: (sc_blocks + i, 0, 0)),
            pl.BlockSpec((TC_BLK, FEATURE_DIM), lambda i: (sc_blocks + i, 0)),
        ],
        out_specs=[
            pl.BlockSpec((NUM_CLASSES, FEATURE_DIM), lambda i: (0, 0)),
            pl.BlockSpec((NUM_CLASSES, 1), lambda i: (0, 0)),
        ],
        out_shape=[
            jax.ShapeDtypeStruct((NUM_CLASSES, FEATURE_DIM), jnp.float32),
            jax.ShapeDtypeStruct((NUM_CLASSES, 1), jnp.float32),
        ],
    )(labels3, features)


def _tc_loss_body(sums_ref, cnts_ref, tc_sums_ref, tc_cnts_ref, out_ref):
    sums = jnp.sum(sums_ref[...], axis=0) + tc_sums_ref[...]
    cnt_all = jnp.sum(cnts_ref[...], axis=0)
    cnt = cnt_all[:, 0:1] + tc_cnts_ref[...]
    present = cnt > 0.0
    cent = jnp.where(present, sums / jnp.maximum(cnt, 1.0), 0.0)
    gram = lax.dot_general(cent, cent, (((1,), (1,)), ((), ())),
                           preferred_element_type=jnp.float32)
    norms = jnp.sum(cent * cent, axis=1, keepdims=True)
    d2 = norms + norms.reshape(1, NUM_CLASSES) - 2.0 * gram
    row = lax.broadcasted_iota(jnp.int32, (NUM_CLASSES, NUM_CLASSES), 0)
    col = lax.broadcasted_iota(jnp.int32, (NUM_CLASSES, NUM_CLASSES), 1)
    valid = (row < col) & present & present.reshape(1, NUM_CLASSES)
    safe = jnp.where(valid, jnp.maximum(d2, 0.0), 1.0)
    terms = jnp.where(valid, jnp.exp(-(jnp.sqrt(safe) / 16.0 + EPS)), 0.0)
    out_ref[...] = jnp.sum(terms)[None, None]


def kernel(features, labels):
    labels32 = labels.astype(jnp.int32)
    sums_p, cnts = _sc_segment_sums(features, labels32)
    tc_sums, tc_cnts = _tc_segment_sums(features, labels32)
    loss = pl.pallas_call(
        _tc_loss_body,
        out_shape=jax.ShapeDtypeStruct((1, 1), jnp.float32),
    )(sums_p, cnts, tc_sums, tc_cnts)
    return loss[0, 0]
